# baseline (speedup 1.0000x reference)
import functools

import jax
import jax.numpy as jnp
from jax import lax
from jax.experimental import pallas as pl
from jax.experimental.pallas import tpu as pltpu
from jax.experimental.pallas import tpu_sc as plsc

_R = 3
_BW = 0.1
_C, _H, _W = 96, 224, 224
_HW = _H * _W
_PAD = 16
_LANES = 16
_NWORK = 32
_CPW = _C // _NWORK
_NVEC = _W // _LANES
_KW = 240
_KOFF = 8


def _body(x_hbm, out_hbm, xbuf, obuf, ke, ks, kse, ksw, sem):
    del sem
    wid = lax.axis_index("s") * 2 + lax.axis_index("c")
    zero16 = jnp.zeros((_LANES,), jnp.float32)
    lane = lax.iota(jnp.int32, _LANES)
    bw = jnp.float32(_BW)

    for k in range(_CPW):
        ch = wid * _CPW + k
        pltpu.sync_copy(x_hbm.at[ch], xbuf.at[pl.ds(_PAD, _HW)])

        for jv in range(_NVEC):
            obuf[pl.ds(jv * _LANES, _LANES)] = zero16
            obuf[pl.ds((_H - 1) * _W + jv * _LANES, _LANES)] = zero16

        for jv in range(_NVEC):
            bx = jv * _LANES + _PAD
            kb = _KOFF + jv * _LANES
            xc = xbuf[pl.ds(bx, _LANES)]
            xd = xbuf[pl.ds(bx + _W, _LANES)]
            xdr = xbuf[pl.ds(bx + _W + 1, _LANES)]
            xdl = xbuf[pl.ds(bx + _W - 1, _LANES)]
            ks[pl.ds(kb, _LANES)] = jnp.maximum(0.0, bw - jnp.abs(xd - xc))
            kse[pl.ds(kb, _LANES)] = jnp.maximum(0.0, bw - jnp.abs(xdr - xc))
            ksw[pl.ds(kb, _LANES)] = jnp.maximum(0.0, bw - jnp.abs(xdl - xc))

        def row_body(i, _):
            s_c = (i & 1) * _KW
            s_p = _KW - s_c
            for jv in range(_NVEC):
                bx = i * _W + jv * _LANES + _PAD
                kb = _KOFF + jv * _LANES
                xc = xbuf[pl.ds(bx, _LANES)]
                xr = xbuf[pl.ds(bx + 1, _LANES)]
                xd = xbuf[pl.ds(bx + _W, _LANES)]
                xdr = xbuf[pl.ds(bx + _W + 1, _LANES)]
                xdl = xbuf[pl.ds(bx + _W - 1, _LANES)]
                vke = jnp.maximum(0.0, bw - jnp.abs(xr - xc))
                vks = jnp.maximum(0.0, bw - jnp.abs(xd - xc))
                vkse = jnp.maximum(0.0, bw - jnp.abs(xdr - xc))
                vksw = jnp.maximum(0.0, bw - jnp.abs(xdl - xc))
                ke[pl.ds(kb, _LANES)] = vke
                ks[pl.ds(s_c + kb, _LANES)] = vks
                kse[pl.ds(s_c + kb, _LANES)] = vkse
                ksw[pl.ds(s_c + kb, _LANES)] = vksw
                kel = ke[pl.ds(kb - 1, _LANES)]
                ksp = ks[pl.ds(s_p + kb, _LANES)]
                ksep = kse[pl.ds(s_p + kb - 1, _LANES)]
                kswp = ksw[pl.ds(s_p + kb + 1, _LANES)]
                acc = (((vke + vks) + (vkse + vksw))
                       + ((kel + ksp) + (ksep + kswp)))
                acc = acc * jnp.float32(1.0 / (_BW * _R * _R)) + jnp.float32(
                    1.0 / (_R * _R))
                if jv == 0:
                    acc = jnp.where(lane >= 1, acc, 0.0)
                if jv == _NVEC - 1:
                    acc = jnp.where(lane <= _LANES - 2, acc, 0.0)
                obuf[pl.ds(i * _W + jv * _LANES, _LANES)] = acc
            return 0

        lax.fori_loop(1, _H - 1, row_body, 0)

        pltpu.sync_copy(obuf, out_hbm.at[ch])


@jax.jit
def _hist_sc(x2d):
    mesh = plsc.VectorSubcoreMesh(core_axis_name="c", subcore_axis_name="s")
    f = pl.kernel(
        _body,
        out_type=jax.ShapeDtypeStruct((_C, _HW), jnp.float32),
        mesh=mesh,
        scratch_types=[
            pltpu.VMEM((_PAD + _HW + _PAD,), jnp.float32),
            pltpu.VMEM((_HW,), jnp.float32),
            pltpu.VMEM((_KW,), jnp.float32),
            pltpu.VMEM((2 * _KW,), jnp.float32),
            pltpu.VMEM((2 * _KW,), jnp.float32),
            pltpu.VMEM((2 * _KW,), jnp.float32),
            pltpu.SemaphoreType.DMA,
        ],
        compiler_params=pltpu.CompilerParams(use_tc_tiling_on_sc=False),
    )
    return f(x2d)


def kernel(input):
    n, sf, c, h, w = input.shape
    x2d = input.reshape(_C, _HW)
    out = _hist_sc(x2d)
    return out.reshape(n, sf, c, h, w)


# speedup vs baseline: 1.6968x; 1.6968x over previous
import jax
import jax.numpy as jnp
from jax import lax
from jax.experimental import pallas as pl
from jax.experimental.pallas import tpu as pltpu
from jax.experimental.pallas import tpu_sc as plsc

_R = 3
_BW = 0.1
_C, _H, _W = 96, 224, 224
_HW = _H * _W
_PAD = 16
_LANES = 16
_NWORK = 32
_CPW = _C // _NWORK
_NVEC = _W // _LANES


def _body(x_hbm, out_hbm, xbuf, obuf, sem):
    del sem
    wid = lax.axis_index("s") * 2 + lax.axis_index("c")
    zero16 = jnp.zeros((_LANES,), jnp.float32)
    lane = lax.iota(jnp.int32, _LANES)

    for k in range(_CPW):
        ch = wid * _CPW + k
        pltpu.sync_copy(x_hbm.at[ch], xbuf.at[pl.ds(_PAD, _HW)])

        for jv in range(_NVEC):
            obuf[pl.ds(jv * _LANES, _LANES)] = zero16
            obuf[pl.ds((_H - 1) * _W + jv * _LANES, _LANES)] = zero16

        for jv in range(_NVEC):
            col0 = jv * _LANES

            @plsc.parallel_loop(1, _H - 1, step=1, unroll=3)
            def row_body(i, col0=col0, jv=jv):
                base = i * _W + col0 + _PAD
                c = xbuf[pl.ds(base, _LANES)]
                acc = jnp.zeros((_LANES,), jnp.float32)
                for di in (-1, 0, 1):
                    for dj in (-1, 0, 1):
                        if di == 0 and dj == 0:
                            continue
                        v = xbuf[pl.ds(base + di * _W + dj, _LANES)]
                        d = (c - v) if (di, dj) < (0, 0) else (v - c)
                        acc = acc + jnp.maximum(0.0, _BW - jnp.abs(d))
                acc = acc * jnp.float32(1.0 / (_BW * _R * _R)) + jnp.float32(
                    1.0 / (_R * _R))
                if jv == 0:
                    acc = jnp.where(lane >= 1, acc, 0.0)
                if jv == _NVEC - 1:
                    acc = jnp.where(lane <= _LANES - 2, acc, 0.0)
                obuf[pl.ds(i * _W + col0, _LANES)] = acc

        pltpu.sync_copy(obuf, out_hbm.at[ch])


@jax.jit
def _hist_sc(x2d):
    mesh = plsc.VectorSubcoreMesh(core_axis_name="c", subcore_axis_name="s")
    f = pl.kernel(
        _body,
        out_type=jax.ShapeDtypeStruct((_C, _HW), jnp.float32),
        mesh=mesh,
        scratch_types=[
            pltpu.VMEM((_PAD + _HW + _PAD,), jnp.float32),
            pltpu.VMEM((_HW,), jnp.float32),
            pltpu.SemaphoreType.DMA,
        ],
        compiler_params=pltpu.CompilerParams(use_tc_tiling_on_sc=False),
    )
    return f(x2d)


def kernel(input):
    n, sf, c, h, w = input.shape
    x2d = input.reshape(_C, _HW)
    out = _hist_sc(x2d)
    return out.reshape(n, sf, c, h, w)


# speedup vs baseline: 2.3936x; 1.4107x over previous
import jax
import jax.numpy as jnp
from jax import lax
from jax.experimental import pallas as pl
from jax.experimental.pallas import tpu as pltpu
from jax.experimental.pallas import tpu_sc as plsc

_R = 3
_BW = 0.1
_C, _H, _W = 96, 224, 224
_HW = _H * _W
_PAD = 16
_LANES = 16
_NWORK = 32
_NSC = 32
_CPW = _NSC // _NWORK
_NVEC = _W // _LANES


def _body(x_hbm, out_hbm, xbuf, obuf, sem):
    del sem
    wid = lax.axis_index("s") * 2 + lax.axis_index("c")
    zero16 = jnp.zeros((_LANES,), jnp.float32)
    lane = lax.iota(jnp.int32, _LANES)

    for k in range(_CPW):
        ch = wid * _CPW + k
        pltpu.sync_copy(x_hbm.at[ch], xbuf.at[pl.ds(_PAD, _HW)])

        for jv in range(_NVEC):
            obuf[pl.ds(jv * _LANES, _LANES)] = zero16
            obuf[pl.ds((_H - 1) * _W + jv * _LANES, _LANES)] = zero16

        for jv in range(_NVEC):
            col0 = jv * _LANES

            @plsc.parallel_loop(1, _H - 1, step=1, unroll=3)
            def row_body(i, col0=col0, jv=jv):
                base = i * _W + col0 + _PAD
                c = xbuf[pl.ds(base, _LANES)]
                acc = jnp.zeros((_LANES,), jnp.float32)
                for di in (-1, 0, 1):
                    for dj in (-1, 0, 1):
                        if di == 0 and dj == 0:
                            continue
                        v = xbuf[pl.ds(base + di * _W + dj, _LANES)]
                        d = (c - v) if (di, dj) < (0, 0) else (v - c)
                        acc = acc + jnp.maximum(0.0, _BW - jnp.abs(d))
                acc = acc * jnp.float32(1.0 / (_BW * _R * _R)) + jnp.float32(
                    1.0 / (_R * _R))
                if jv == 0:
                    acc = jnp.where(lane >= 1, acc, 0.0)
                if jv == _NVEC - 1:
                    acc = jnp.where(lane <= _LANES - 2, acc, 0.0)
                obuf[pl.ds(i * _W + col0, _LANES)] = acc

        pltpu.sync_copy(obuf, out_hbm.at[ch])


def _hist_sc(x2d):
    mesh = plsc.VectorSubcoreMesh(core_axis_name="c", subcore_axis_name="s")
    f = pl.kernel(
        _body,
        out_type=jax.ShapeDtypeStruct((_NSC, _HW), jnp.float32),
        mesh=mesh,
        scratch_types=[
            pltpu.VMEM((_PAD + _HW + _PAD,), jnp.float32),
            pltpu.VMEM((_HW,), jnp.float32),
            pltpu.SemaphoreType.DMA,
        ],
        compiler_params=pltpu.CompilerParams(use_tc_tiling_on_sc=False),
    )
    return f(x2d)


_BC = 8


def _tc_body(x_ref, o_ref):
    x = x_ref[...]
    rows = {di: jnp.roll(x, -di, axis=1) if di else x for di in (-1, 0, 1)}
    acc = jnp.zeros_like(x)
    for di in (-1, 0, 1):
        for dj in (-1, 0, 1):
            if di == 0 and dj == 0:
                continue
            v = jnp.roll(rows[di], -dj, axis=2) if dj else rows[di]
            acc = acc + jnp.maximum(0.0, _BW - jnp.abs(v - x))
    acc = acc * jnp.float32(1.0 / (_BW * _R * _R)) + jnp.float32(1.0 / (_R * _R))
    row = lax.broadcasted_iota(jnp.int32, x.shape, 1)
    col = lax.broadcasted_iota(jnp.int32, x.shape, 2)
    interior = ((row >= 1) & (row <= _H - 2)) & ((col >= 1) & (col <= _W - 2))
    o_ref[...] = jnp.where(interior, acc, 0.0)


def _hist_tc(x):
    ct = x.shape[0]
    return pl.pallas_call(
        _tc_body,
        out_shape=jax.ShapeDtypeStruct((ct, _H, _W), jnp.float32),
        grid=(ct // _BC,),
        in_specs=[pl.BlockSpec((_BC, _H, _W), lambda i: (i, 0, 0))],
        out_specs=pl.BlockSpec((_BC, _H, _W), lambda i: (i, 0, 0)),
    )(x)


@jax.jit
def _hist(x3):
    sc_out = _hist_sc(x3[:_NSC].reshape(_NSC, _HW)).reshape(_NSC, _H, _W)
    tc_out = _hist_tc(x3[_NSC:])
    return jnp.concatenate([sc_out, tc_out], axis=0)


def kernel(input):
    n, sf, c, h, w = input.shape
    out = _hist(input.reshape(c, h, w))
    return out.reshape(n, sf, c, h, w)
